# baseline (speedup 1.0000x reference)
import functools

import jax
import jax.numpy as jnp
from jax import lax
from jax.experimental import pallas as pl
from jax.experimental.pallas import tpu as pltpu
from jax.experimental.pallas import tpu_sc as plsc

Q = 32
D = 64
N = 1_000_000
BLK = 8192
NB = (N + BLK - 1) // BLK
NPAD = NB * BLK
NCHUNK = NB * 128
TOP_K = 10
CAND = TOP_K * (BLK // 128)
TEMPERATURE = 0.07
NEG = float("-inf")
BIG = 2 ** 30

NUM_SC_CORES = 2
NUM_SC_SUBCORES = 16

_SC_PARAMS = pltpu.CompilerParams(needs_layout_passes=False)


def _extract(vec, lane, j, fill):
    return jnp.max(jnp.where(lane == j, vec, fill))


def _k1_body(q_ref, k_ref, sims_ref, bq_ref, lq_ref, cmax_ref):
    b = pl.program_id(0)
    q = q_ref[...]
    qn = q / (jnp.sqrt(jnp.sum(q * q, axis=1, keepdims=True)) + 1e-8)
    qh = qn.astype(jnp.bfloat16)

    kb = k_ref[...]
    ksq = kb * kb
    s2 = jnp.sum(ksq.T, axis=0, keepdims=True)
    s2col = s2.reshape(BLK, 1)
    kh = (kb / (jnp.sqrt(s2col) + 1e-8)).astype(jnp.bfloat16)
    sim = lax.dot_general(qh, kh, (((1,), (1,)), ((), ())),
                          preferred_element_type=jnp.float32)

    col = b * BLK + lax.broadcasted_iota(jnp.int32, (1, BLK), 1)
    sim = jnp.where(col < N, sim, NEG)
    sims_ref[...] = sim

    m = sim[:, 0:128]
    for c in range(1, BLK // 128):
        m = jnp.maximum(m, sim[:, c * 128:(c + 1) * 128])
    cmax_ref[:, pl.ds(pl.multiple_of(b * 128, 128), 128)] = m

    @pl.when(b == NB - 1)
    def _select():
        cm = cmax_ref[...]
        ids = lax.broadcasted_iota(jnp.int32, (Q, NCHUNK), 1)
        bq_ref[...] = jnp.zeros((Q, 16), jnp.int32)
        lq_ref[...] = jnp.zeros((Q, 16), jnp.int32)
        for r in range(TOP_K):
            mval = jnp.max(cm, axis=1, keepdims=True)
            sel = jnp.min(jnp.where(cm == mval, ids, BIG),
                          axis=1, keepdims=True)
            cm = jnp.where(ids == sel, NEG, cm)
            bq_ref[:, r:r + 1] = sel >> 7
            lq_ref[:, r:r + 1] = sel & 127


def _run_k1(queries, keys):
    return pl.pallas_call(
        _k1_body,
        grid=(NB,),
        in_specs=[
            pl.BlockSpec((Q, D), lambda b: (0, 0)),
            pl.BlockSpec((BLK, D), lambda b: (b, 0)),
        ],
        out_specs=[
            pl.BlockSpec((Q, BLK), lambda b: (0, b)),
            pl.BlockSpec((Q, 16), lambda b: (0, 0)),
            pl.BlockSpec((Q, 16), lambda b: (0, 0)),
        ],
        out_shape=[
            jax.ShapeDtypeStruct((Q, NPAD), jnp.float32),
            jax.ShapeDtypeStruct((Q, 16), jnp.int32),
            jax.ShapeDtypeStruct((Q, 16), jnp.int32),
        ],
        scratch_shapes=[pltpu.VMEM((Q, NCHUNK), jnp.float32)],
    )(queries, keys)


def _run_k345(sims, bq16, lq16, token_bank):
    mesh = plsc.VectorSubcoreMesh(core_axis_name="c", subcore_axis_name="s")

    @functools.partial(
        pl.kernel,
        out_type=[
            jax.ShapeDtypeStruct((Q, D), jnp.float32),
            jax.ShapeDtypeStruct((Q, 16), jnp.int32),
        ],
        mesh=mesh,
        scratch_types=[
            pltpu.VMEM((16,), jnp.int32),
            pltpu.VMEM((16,), jnp.int32),
            pltpu.VMEM((TOP_K * BLK,), jnp.float32),
            pltpu.VMEM((16, D), jnp.float32),
            pltpu.VMEM((16,), jnp.int32),
            pltpu.VMEM((D,), jnp.float32),
            pltpu.SemaphoreType.DMA,
        ],
        compiler_params=_SC_PARAMS,
    )
    def k345(sims_hbm, bq_hbm, lq_hbm, tok_hbm, prompt_hbm, idx_hbm,
             b_v, l_v, rows_v, toks_v, iout_v, acc_v, sem):
        qid = lax.axis_index("s") * NUM_SC_CORES + lax.axis_index("c")
        pltpu.sync_copy(bq_hbm.at[qid], b_v)
        pltpu.sync_copy(lq_hbm.at[qid], l_v)
        bvec = b_v[...]
        lvec = l_v[...]
        lane = lax.iota(jnp.int32, 16)

        copies = []
        for r in range(TOP_K):
            br = _extract(bvec, lane, r, -1)
            copies.append(pltpu.async_copy(
                sims_hbm.at[qid, pl.ds(br * BLK, BLK)],
                rows_v.at[pl.ds(r * BLK, BLK)], sem))
        for cp in copies:
            cp.wait()

        cand = []
        pos = []
        for r in range(TOP_K):
            br = _extract(bvec, lane, r, -1)
            lr = _extract(lvec, lane, r, -1)
            for c4 in range(4):
                off = lr + 2048 * c4 + 128 * lane
                cand.append(plsc.load_gather(rows_v, [r * BLK + off]))
                pos.append(br * BLK + off)

        vals16 = jnp.full((16,), NEG, jnp.float32)
        idxs16 = jnp.zeros((16,), jnp.int32)
        for r in range(TOP_K):
            mvec = cand[0]
            for g in range(1, len(cand)):
                mvec = jnp.maximum(mvec, cand[g])
            m = jnp.max(mvec)
            pvec = jnp.where(cand[0] == m, pos[0], BIG)
            for g in range(1, len(cand)):
                pvec = jnp.minimum(pvec, jnp.where(cand[g] == m, pos[g], BIG))
            pi = jnp.min(pvec)
            vals16 = jnp.where(lane == r, m, vals16)
            idxs16 = jnp.where(lane == r, pi, idxs16)
            cand = [jnp.where(pos[g] == pi, NEG, cand[g])
                    for g in range(len(cand))]

        t = vals16 / TEMPERATURE
        e = jnp.exp(t - jnp.max(t))
        w = e / jnp.sum(e)

        iout_v[...] = idxs16
        pltpu.sync_copy(iout_v, idx_hbm.at[qid])

        copies = []
        for j in range(TOP_K):
            rj = _extract(idxs16, lane, j, -1)
            copies.append(pltpu.async_copy(tok_hbm.at[rj], toks_v.at[j], sem))
        for cp in copies:
            cp.wait()
        for c in range(D // 16):
            acc = jnp.zeros((16,), jnp.float32)
            for j in range(TOP_K):
                wj = _extract(w, lane, j, NEG)
                acc = acc + toks_v[j, pl.ds(c * 16, 16)] * wj
            acc_v[pl.ds(c * 16, 16)] = acc
        pltpu.sync_copy(acc_v, prompt_hbm.at[qid])

    return k345(sims, bq16, lq16, token_bank)


def kernel(queries, keys, token_bank):
    sims, bq16, lq16 = _run_k1(queries, keys)
    prompt, idx16 = _run_k345(sims, bq16, lq16, token_bank)
    top_idx = idx16[:, :TOP_K]
    return prompt, top_idx


# speedup vs baseline: 1.9878x; 1.9878x over previous
import functools

import jax
import jax.numpy as jnp
from jax import lax
from jax.experimental import pallas as pl
from jax.experimental.pallas import tpu as pltpu
from jax.experimental.pallas import tpu_sc as plsc

Q = 32
D = 64
N = 1_000_000
BLK = 8192
NB = (N + BLK - 1) // BLK
NPAD = NB * BLK
NCHUNK = NB * 128
TOP_K = 10
CAND = TOP_K * (BLK // 128)
TEMPERATURE = 0.07
NEG = float("-inf")
BIG = 2 ** 30

NUM_SC_CORES = 2
NUM_SC_SUBCORES = 16

_SC_PARAMS = pltpu.CompilerParams(needs_layout_passes=False)


def _extract(vec, lane, j, fill):
    return jnp.max(jnp.where(lane == j, vec, fill))


def _k1_body(q_ref, kt_ref, sims_ref, bq_ref, lq_ref, cmax_ref):
    b = pl.program_id(0)
    q = q_ref[...]
    qn = q / (jnp.sqrt(jnp.sum(q * q, axis=1, keepdims=True)) + 1e-8)
    qh = qn.astype(jnp.bfloat16)

    kt = kt_ref[...]
    ksq = kt * kt
    s2 = jnp.sum(ksq, axis=0, keepdims=True)
    kh = (kt / (jnp.sqrt(s2) + 1e-8)).astype(jnp.bfloat16)
    sim = lax.dot_general(qh, kh, (((1,), (0,)), ((), ())),
                          preferred_element_type=jnp.float32)

    col = b * BLK + lax.broadcasted_iota(jnp.int32, (1, BLK), 1)
    sim = jnp.where(col < N, sim, NEG)
    sims_ref[...] = sim

    m = sim[:, 0:128]
    for c in range(1, BLK // 128):
        m = jnp.maximum(m, sim[:, c * 128:(c + 1) * 128])
    cmax_ref[:, pl.ds(pl.multiple_of(b * 128, 128), 128)] = m

    @pl.when(b == NB - 1)
    def _select():
        cm = cmax_ref[...]
        ids = lax.broadcasted_iota(jnp.int32, (Q, NCHUNK), 1)
        bq_ref[...] = jnp.zeros((Q, 16), jnp.int32)
        lq_ref[...] = jnp.zeros((Q, 16), jnp.int32)
        for r in range(TOP_K):
            mval = jnp.max(cm, axis=1, keepdims=True)
            sel = jnp.min(jnp.where(cm == mval, ids, BIG),
                          axis=1, keepdims=True)
            cm = jnp.where(ids == sel, NEG, cm)
            bq_ref[:, r:r + 1] = sel >> 7
            lq_ref[:, r:r + 1] = sel & 127


def _run_k1(queries, keys_t):
    return pl.pallas_call(
        _k1_body,
        grid=(NB,),
        in_specs=[
            pl.BlockSpec((Q, D), lambda b: (0, 0)),
            pl.BlockSpec((D, BLK), lambda b: (0, b)),
        ],
        out_specs=[
            pl.BlockSpec((Q, BLK), lambda b: (0, b)),
            pl.BlockSpec((Q, 16), lambda b: (0, 0)),
            pl.BlockSpec((Q, 16), lambda b: (0, 0)),
        ],
        out_shape=[
            jax.ShapeDtypeStruct((Q, NPAD), jnp.float32),
            jax.ShapeDtypeStruct((Q, 16), jnp.int32),
            jax.ShapeDtypeStruct((Q, 16), jnp.int32),
        ],
        scratch_shapes=[pltpu.VMEM((Q, NCHUNK), jnp.float32)],
    )(queries, keys_t)


def _run_k345(sims, bq16, lq16, token_bank):
    mesh = plsc.VectorSubcoreMesh(core_axis_name="c", subcore_axis_name="s")

    @functools.partial(
        pl.kernel,
        out_type=[
            jax.ShapeDtypeStruct((Q, D), jnp.float32),
            jax.ShapeDtypeStruct((Q, 16), jnp.int32),
        ],
        mesh=mesh,
        scratch_types=[
            pltpu.VMEM((16,), jnp.int32),
            pltpu.VMEM((16,), jnp.int32),
            pltpu.VMEM((TOP_K * BLK,), jnp.float32),
            pltpu.VMEM((16, D), jnp.float32),
            pltpu.VMEM((16,), jnp.int32),
            pltpu.VMEM((D,), jnp.float32),
            pltpu.SemaphoreType.DMA,
        ],
        compiler_params=_SC_PARAMS,
    )
    def k345(sims_hbm, bq_hbm, lq_hbm, tok_hbm, prompt_hbm, idx_hbm,
             b_v, l_v, rows_v, toks_v, iout_v, acc_v, sem):
        qid = lax.axis_index("s") * NUM_SC_CORES + lax.axis_index("c")
        pltpu.sync_copy(bq_hbm.at[qid], b_v)
        pltpu.sync_copy(lq_hbm.at[qid], l_v)
        bvec = b_v[...]
        lvec = l_v[...]
        lane = lax.iota(jnp.int32, 16)

        copies = []
        for r in range(TOP_K):
            br = _extract(bvec, lane, r, -1)
            copies.append(pltpu.async_copy(
                sims_hbm.at[qid, pl.ds(br * BLK, BLK)],
                rows_v.at[pl.ds(r * BLK, BLK)], sem))
        for cp in copies:
            cp.wait()

        cand = []
        pos = []
        for r in range(TOP_K):
            br = _extract(bvec, lane, r, -1)
            lr = _extract(lvec, lane, r, -1)
            for c4 in range(4):
                off = lr + 2048 * c4 + 128 * lane
                cand.append(plsc.load_gather(rows_v, [r * BLK + off]))
                pos.append(br * BLK + off)

        vals16 = jnp.full((16,), NEG, jnp.float32)
        idxs16 = jnp.zeros((16,), jnp.int32)
        for r in range(TOP_K):
            mvec = cand[0]
            for g in range(1, len(cand)):
                mvec = jnp.maximum(mvec, cand[g])
            m = jnp.max(mvec)
            pvec = jnp.where(cand[0] == m, pos[0], BIG)
            for g in range(1, len(cand)):
                pvec = jnp.minimum(pvec, jnp.where(cand[g] == m, pos[g], BIG))
            pi = jnp.min(pvec)
            vals16 = jnp.where(lane == r, m, vals16)
            idxs16 = jnp.where(lane == r, pi, idxs16)
            cand = [jnp.where(pos[g] == pi, NEG, cand[g])
                    for g in range(len(cand))]

        t = vals16 / TEMPERATURE
        e = jnp.exp(t - jnp.max(t))
        w = e / jnp.sum(e)

        iout_v[...] = idxs16
        pltpu.sync_copy(iout_v, idx_hbm.at[qid])

        copies = []
        for j in range(TOP_K):
            rj = _extract(idxs16, lane, j, -1)
            copies.append(pltpu.async_copy(tok_hbm.at[rj], toks_v.at[j], sem))
        for cp in copies:
            cp.wait()
        for c in range(D // 16):
            acc = jnp.zeros((16,), jnp.float32)
            for j in range(TOP_K):
                wj = _extract(w, lane, j, NEG)
                acc = acc + toks_v[j, pl.ds(c * 16, 16)] * wj
            acc_v[pl.ds(c * 16, 16)] = acc
        pltpu.sync_copy(acc_v, prompt_hbm.at[qid])

    return k345(sims, bq16, lq16, token_bank)


def kernel(queries, keys, token_bank):
    sims, bq16, lq16 = _run_k1(queries, keys.T)
    prompt, idx16 = _run_k345(sims, bq16, lq16, token_bank)
    top_idx = idx16[:, :TOP_K]
    return prompt, top_idx


# speedup vs baseline: 5.1392x; 2.5853x over previous
import functools

import jax
import jax.numpy as jnp
from jax import lax
from jax.experimental import pallas as pl
from jax.experimental.pallas import tpu as pltpu
from jax.experimental.pallas import tpu_sc as plsc

Q = 32
D = 64
N = 1_000_000
BLK = 8192
NB = (N + BLK - 1) // BLK
NPAD = NB * BLK
NCHUNK = NB * 128
TOP_K = 10
CAND = TOP_K * (BLK // 128)
TEMPERATURE = 0.07
NEG = float("-inf")
BIG = 2 ** 30

NUM_SC_CORES = 2
NUM_SC_SUBCORES = 16

_SC_PARAMS = pltpu.CompilerParams(needs_layout_passes=False)


def _extract(vec, lane, j, fill):
    return jnp.max(jnp.where(lane == j, vec, fill))


def _k1_body(q_ref, kt_ref, sims_ref, bq_ref, lq_ref, cmax_ref):
    b = pl.program_id(0)
    q = q_ref[...]
    qn = q / (jnp.sqrt(jnp.sum(q * q, axis=1, keepdims=True)) + 1e-8)
    qh = qn.astype(jnp.bfloat16)

    kt = kt_ref[...]
    ksq = kt * kt
    s2 = jnp.sum(ksq, axis=0, keepdims=True)
    kh = (kt / (jnp.sqrt(s2) + 1e-8)).astype(jnp.bfloat16)
    sim = lax.dot_general(qh, kh, (((1,), (0,)), ((), ())),
                          preferred_element_type=jnp.float32)

    col = b * BLK + lax.broadcasted_iota(jnp.int32, (1, BLK), 1)
    sim = jnp.where(col < N, sim, NEG)
    sims_ref[...] = sim

    m = sim[:, 0:128]
    for c in range(1, BLK // 128):
        m = jnp.maximum(m, sim[:, c * 128:(c + 1) * 128])
    cmax_ref[:, pl.ds(pl.multiple_of(b * 128, 128), 128)] = m

    @pl.when(b == NB - 1)
    def _select():
        cm = cmax_ref[...]
        ids = lax.broadcasted_iota(jnp.int32, (Q, NCHUNK), 1)
        bq_ref[...] = jnp.zeros((Q, 16), jnp.int32)
        lq_ref[...] = jnp.zeros((Q, 16), jnp.int32)
        for r in range(TOP_K):
            mval = jnp.max(cm, axis=1, keepdims=True)
            sel = jnp.min(jnp.where(cm == mval, ids, BIG),
                          axis=1, keepdims=True)
            cm = jnp.where(ids == sel, NEG, cm)
            bq_ref[:, r:r + 1] = sel >> 7
            lq_ref[:, r:r + 1] = sel & 127


def _run_k1(queries, keys_t):
    return pl.pallas_call(
        _k1_body,
        grid=(NB,),
        in_specs=[
            pl.BlockSpec((Q, D), lambda b: (0, 0)),
            pl.BlockSpec((D, BLK), lambda b: (0, b)),
        ],
        out_specs=[
            pl.BlockSpec((Q, BLK), lambda b: (0, b)),
            pl.BlockSpec((Q, 16), lambda b: (0, 0)),
            pl.BlockSpec((Q, 16), lambda b: (0, 0)),
        ],
        out_shape=[
            jax.ShapeDtypeStruct((Q, NPAD), jnp.float32),
            jax.ShapeDtypeStruct((Q, 16), jnp.int32),
            jax.ShapeDtypeStruct((Q, 16), jnp.int32),
        ],
        scratch_shapes=[pltpu.VMEM((Q, NCHUNK), jnp.float32)],
    )(queries, keys_t)


def _run_k345(sims, bq16, lq16, token_bank):
    mesh = plsc.VectorSubcoreMesh(core_axis_name="c", subcore_axis_name="s")

    @functools.partial(
        pl.kernel,
        out_type=[
            jax.ShapeDtypeStruct((Q, D), jnp.float32),
            jax.ShapeDtypeStruct((Q, 16), jnp.int32),
        ],
        mesh=mesh,
        scratch_types=[
            pltpu.VMEM((16,), jnp.int32),
            pltpu.VMEM((16,), jnp.int32),
            pltpu.VMEM((5 * BLK,), jnp.float32),
            pltpu.VMEM((TOP_K, D, 128), jnp.float32),
            pltpu.VMEM((16,), jnp.int32),
            pltpu.VMEM((D,), jnp.float32),
            pltpu.SemaphoreType.DMA,
        ],
        compiler_params=_SC_PARAMS,
    )
    def k345(sims_hbm, bq_hbm, lq_hbm, tokt_hbm, prompt_hbm, idx_hbm,
             b_v, l_v, rows_v, tokslab_v, iout_v, acc_v, sem):
        qid = lax.axis_index("s") * NUM_SC_CORES + lax.axis_index("c")
        pltpu.sync_copy(bq_hbm.at[qid], b_v)
        pltpu.sync_copy(lq_hbm.at[qid], l_v)
        bvec = b_v[...]
        lvec = l_v[...]
        lane = lax.iota(jnp.int32, 16)

        cand = []
        pos = []
        for wave in range(2):
            copies = []
            for s in range(5):
                r = wave * 5 + s
                br = _extract(bvec, lane, r, -1)
                copies.append(pltpu.async_copy(
                    sims_hbm.at[qid, pl.ds(br * BLK, BLK)],
                    rows_v.at[pl.ds(s * BLK, BLK)], sem))
            for cp in copies:
                cp.wait()
            for s in range(5):
                r = wave * 5 + s
                br = _extract(bvec, lane, r, -1)
                lr = _extract(lvec, lane, r, -1)
                for c4 in range(4):
                    off = lr + 2048 * c4 + 128 * lane
                    cand.append(plsc.load_gather(rows_v, [s * BLK + off]))
                    pos.append(br * BLK + off)

        vals16 = jnp.full((16,), NEG, jnp.float32)
        idxs16 = jnp.zeros((16,), jnp.int32)
        for r in range(TOP_K):
            mvec = cand[0]
            for g in range(1, len(cand)):
                mvec = jnp.maximum(mvec, cand[g])
            m = jnp.max(mvec)
            pvec = jnp.where(cand[0] == m, pos[0], BIG)
            for g in range(1, len(cand)):
                pvec = jnp.minimum(pvec, jnp.where(cand[g] == m, pos[g], BIG))
            pi = jnp.min(pvec)
            vals16 = jnp.where(lane == r, m, vals16)
            idxs16 = jnp.where(lane == r, pi, idxs16)
            cand = [jnp.where(pos[g] == pi, NEG, cand[g])
                    for g in range(len(cand))]

        t = vals16 / TEMPERATURE
        e = jnp.exp(t - jnp.max(t))
        w = e / jnp.sum(e)

        iout_v[...] = idxs16
        pltpu.sync_copy(iout_v, idx_hbm.at[qid])

        copies = []
        for j in range(TOP_K):
            rj = _extract(idxs16, lane, j, -1)
            off = pl.multiple_of((rj >> 7) * 128, 128)
            copies.append(pltpu.async_copy(
                tokt_hbm.at[:, pl.ds(off, 128)], tokslab_v.at[j], sem))
        for cp in copies:
            cp.wait()
        for c in range(D // 16):
            acc = jnp.zeros((16,), jnp.float32)
            for j in range(TOP_K):
                rj = _extract(idxs16, lane, j, -1)
                rm = jnp.full((16,), rj & 127, jnp.int32)
                wj = _extract(w, lane, j, NEG)
                v = plsc.load_gather(
                    tokslab_v,
                    [jnp.full((16,), j, jnp.int32), lane + 16 * c, rm])
                acc = acc + v * wj
            acc_v[pl.ds(c * 16, 16)] = acc
        pltpu.sync_copy(acc_v, prompt_hbm.at[qid])

    return k345(sims, bq16, lq16, token_bank)


def kernel(queries, keys, token_bank):
    sims, bq16, lq16 = _run_k1(queries, keys.T)
    prompt, idx16 = _run_k345(sims, bq16, lq16, token_bank.T)
    top_idx = idx16[:, :TOP_K]
    return prompt, top_idx


# speedup vs baseline: 5.1405x; 1.0002x over previous
import functools

import jax
import jax.numpy as jnp
from jax import lax
from jax.experimental import pallas as pl
from jax.experimental.pallas import tpu as pltpu
from jax.experimental.pallas import tpu_sc as plsc

Q = 32
D = 64
N = 1_000_000
BLK = 8192
NB = (N + BLK - 1) // BLK
NPAD = NB * BLK
NCHUNK = NB * 128
TOP_K = 10
CAND = TOP_K * (BLK // 128)
TEMPERATURE = 0.07
NEG = float("-inf")
BIG = 2 ** 30

NUM_SC_CORES = 2
NUM_SC_SUBCORES = 16

_SC_PARAMS = pltpu.CompilerParams(needs_layout_passes=False)


def _extract(vec, lane, j, fill):
    return jnp.max(jnp.where(lane == j, vec, fill))


def _k1_body(q_ref, kt_ref, sims_ref, bq_ref, lq_ref, cmax_ref):
    b = pl.program_id(0)
    q = q_ref[...]
    qn = q / (jnp.sqrt(jnp.sum(q * q, axis=1, keepdims=True)) + 1e-8)
    qh = qn.astype(jnp.bfloat16)

    kt = kt_ref[...]
    ksq = kt * kt
    s2 = jnp.sum(ksq, axis=0, keepdims=True)
    rinv = 1.0 / (jnp.sqrt(s2) + 1e-8)
    kh = (kt * rinv).astype(jnp.bfloat16)
    sim = lax.dot_general(qh, kh, (((1,), (0,)), ((), ())),
                          preferred_element_type=jnp.float32)

    col = b * BLK + lax.broadcasted_iota(jnp.int32, (1, BLK), 1)
    sim = jnp.where(col < N, sim, NEG)
    sims_ref[...] = sim

    m = sim[:, 0:128]
    for c in range(1, BLK // 128):
        m = jnp.maximum(m, sim[:, c * 128:(c + 1) * 128])
    cmax_ref[:, pl.ds(pl.multiple_of(b * 128, 128), 128)] = m

    @pl.when(b == NB - 1)
    def _select():
        cm = cmax_ref[...]
        ids = lax.broadcasted_iota(jnp.int32, (Q, NCHUNK), 1)
        bq_ref[...] = jnp.zeros((Q, 16), jnp.int32)
        lq_ref[...] = jnp.zeros((Q, 16), jnp.int32)
        for r in range(TOP_K):
            mval = jnp.max(cm, axis=1, keepdims=True)
            sel = jnp.min(jnp.where(cm == mval, ids, BIG),
                          axis=1, keepdims=True)
            cm = jnp.where(ids == sel, NEG, cm)
            bq_ref[:, r:r + 1] = sel >> 7
            lq_ref[:, r:r + 1] = sel & 127


def _run_k1(queries, keys_t):
    return pl.pallas_call(
        _k1_body,
        grid=(NB,),
        in_specs=[
            pl.BlockSpec((Q, D), lambda b: (0, 0)),
            pl.BlockSpec((D, BLK), lambda b: (0, b)),
        ],
        out_specs=[
            pl.BlockSpec((Q, BLK), lambda b: (0, b)),
            pl.BlockSpec((Q, 16), lambda b: (0, 0)),
            pl.BlockSpec((Q, 16), lambda b: (0, 0)),
        ],
        out_shape=[
            jax.ShapeDtypeStruct((Q, NPAD), jnp.float32),
            jax.ShapeDtypeStruct((Q, 16), jnp.int32),
            jax.ShapeDtypeStruct((Q, 16), jnp.int32),
        ],
        scratch_shapes=[pltpu.VMEM((Q, NCHUNK), jnp.float32)],
    )(queries, keys_t)


def _run_k345(sims, bq16, lq16, token_bank):
    mesh = plsc.VectorSubcoreMesh(core_axis_name="c", subcore_axis_name="s")

    @functools.partial(
        pl.kernel,
        out_type=[
            jax.ShapeDtypeStruct((Q, D), jnp.float32),
            jax.ShapeDtypeStruct((Q, 16), jnp.int32),
        ],
        mesh=mesh,
        scratch_types=[
            pltpu.VMEM((16,), jnp.int32),
            pltpu.VMEM((16,), jnp.int32),
            pltpu.VMEM((5 * BLK,), jnp.float32),
            pltpu.VMEM((TOP_K, D, 128), jnp.float32),
            pltpu.VMEM((16,), jnp.int32),
            pltpu.VMEM((D,), jnp.float32),
            pltpu.SemaphoreType.DMA,
        ],
        compiler_params=_SC_PARAMS,
    )
    def k345(sims_hbm, bq_hbm, lq_hbm, tokt_hbm, prompt_hbm, idx_hbm,
             b_v, l_v, rows_v, tokslab_v, iout_v, acc_v, sem):
        qid = lax.axis_index("s") * NUM_SC_CORES + lax.axis_index("c")
        pltpu.sync_copy(bq_hbm.at[qid], b_v)
        pltpu.sync_copy(lq_hbm.at[qid], l_v)
        bvec = b_v[...]
        lvec = l_v[...]
        lane = lax.iota(jnp.int32, 16)

        cand = []
        pos = []
        for wave in range(2):
            copies = []
            for s in range(5):
                r = wave * 5 + s
                br = _extract(bvec, lane, r, -1)
                copies.append(pltpu.async_copy(
                    sims_hbm.at[qid, pl.ds(br * BLK, BLK)],
                    rows_v.at[pl.ds(s * BLK, BLK)], sem))
            for cp in copies:
                cp.wait()
            for s in range(5):
                r = wave * 5 + s
                br = _extract(bvec, lane, r, -1)
                lr = _extract(lvec, lane, r, -1)
                for c4 in range(4):
                    off = lr + 2048 * c4 + 128 * lane
                    cand.append(plsc.load_gather(rows_v, [s * BLK + off]))
                    pos.append(br * BLK + off)

        vals16 = jnp.full((16,), NEG, jnp.float32)
        idxs16 = jnp.zeros((16,), jnp.int32)
        for r in range(TOP_K):
            mvec = cand[0]
            for g in range(1, len(cand)):
                mvec = jnp.maximum(mvec, cand[g])
            m = jnp.max(mvec)
            pvec = jnp.where(cand[0] == m, pos[0], BIG)
            for g in range(1, len(cand)):
                pvec = jnp.minimum(pvec, jnp.where(cand[g] == m, pos[g], BIG))
            pi = jnp.min(pvec)
            vals16 = jnp.where(lane == r, m, vals16)
            idxs16 = jnp.where(lane == r, pi, idxs16)
            cand = [jnp.where(pos[g] == pi, NEG, cand[g])
                    for g in range(len(cand))]

        t = vals16 / TEMPERATURE
        e = jnp.exp(t - jnp.max(t))
        w = e / jnp.sum(e)

        iout_v[...] = idxs16
        pltpu.sync_copy(iout_v, idx_hbm.at[qid])

        copies = []
        for j in range(TOP_K):
            rj = _extract(idxs16, lane, j, -1)
            off = pl.multiple_of((rj >> 7) * 128, 128)
            copies.append(pltpu.async_copy(
                tokt_hbm.at[:, pl.ds(off, 128)], tokslab_v.at[j], sem))
        for cp in copies:
            cp.wait()
        for c in range(D // 16):
            acc = jnp.zeros((16,), jnp.float32)
            for j in range(TOP_K):
                rj = _extract(idxs16, lane, j, -1)
                rm = jnp.full((16,), rj & 127, jnp.int32)
                wj = _extract(w, lane, j, NEG)
                v = plsc.load_gather(
                    tokslab_v,
                    [jnp.full((16,), j, jnp.int32), lane + 16 * c, rm])
                acc = acc + v * wj
            acc_v[pl.ds(c * 16, 16)] = acc
        pltpu.sync_copy(acc_v, prompt_hbm.at[qid])

    return k345(sims, bq16, lq16, token_bank)


def kernel(queries, keys, token_bank):
    sims, bq16, lq16 = _run_k1(queries, keys.T)
    prompt, idx16 = _run_k345(sims, bq16, lq16, token_bank.T)
    top_idx = idx16[:, :TOP_K]
    return prompt, top_idx


# speedup vs baseline: 5.8879x; 1.1454x over previous
import functools

import jax
import jax.numpy as jnp
from jax import lax
from jax.experimental import pallas as pl
from jax.experimental.pallas import tpu as pltpu
from jax.experimental.pallas import tpu_sc as plsc

Q = 32
D = 64
N = 1_000_000
BLK = 16384
NB = (N + BLK - 1) // BLK
NPAD = NB * BLK
NCHUNK = NB * 128
TOP_K = 10
CAND = TOP_K * (BLK // 128)
TEMPERATURE = 0.07
NEG = float("-inf")
BIG = 2 ** 30

NUM_SC_CORES = 2
NUM_SC_SUBCORES = 16

_SC_PARAMS = pltpu.CompilerParams(needs_layout_passes=False)


def _extract(vec, lane, j, fill):
    return jnp.max(jnp.where(lane == j, vec, fill))


def _k1_body(q_ref, kt_ref, sims_ref, bq_ref, lq_ref, cmax_ref):
    b = pl.program_id(0)
    q = q_ref[...]
    qn = q / (jnp.sqrt(jnp.sum(q * q, axis=1, keepdims=True)) + 1e-8)
    qh = qn.astype(jnp.bfloat16)

    kt = kt_ref[...]
    ksq = kt * kt
    s2 = jnp.sum(ksq, axis=0, keepdims=True)
    rinv = 1.0 / (jnp.sqrt(s2) + 1e-8)
    kh = (kt * rinv).astype(jnp.bfloat16)
    sim = lax.dot_general(qh, kh, (((1,), (0,)), ((), ())),
                          preferred_element_type=jnp.float32)

    col = b * BLK + lax.broadcasted_iota(jnp.int32, (1, BLK), 1)
    sim = jnp.where(col < N, sim, NEG)
    sims_ref[...] = sim

    m = sim[:, 0:128]
    for c in range(1, BLK // 128):
        m = jnp.maximum(m, sim[:, c * 128:(c + 1) * 128])
    cmax_ref[:, pl.ds(pl.multiple_of(b * 128, 128), 128)] = m

    @pl.when(b == NB - 1)
    def _select():
        cm = cmax_ref[...]
        ids = lax.broadcasted_iota(jnp.int32, (Q, NCHUNK), 1)
        bq_ref[...] = jnp.zeros((Q, 16), jnp.int32)
        lq_ref[...] = jnp.zeros((Q, 16), jnp.int32)
        for r in range(TOP_K):
            mval = jnp.max(cm, axis=1, keepdims=True)
            sel = jnp.min(jnp.where(cm == mval, ids, BIG),
                          axis=1, keepdims=True)
            cm = jnp.where(ids == sel, NEG, cm)
            bq_ref[:, r:r + 1] = sel >> 7
            lq_ref[:, r:r + 1] = sel & 127


def _run_k1(queries, keys_t):
    return pl.pallas_call(
        _k1_body,
        grid=(NB,),
        in_specs=[
            pl.BlockSpec((Q, D), lambda b: (0, 0)),
            pl.BlockSpec((D, BLK), lambda b: (0, b)),
        ],
        out_specs=[
            pl.BlockSpec((Q, BLK), lambda b: (0, b)),
            pl.BlockSpec((Q, 16), lambda b: (0, 0)),
            pl.BlockSpec((Q, 16), lambda b: (0, 0)),
        ],
        out_shape=[
            jax.ShapeDtypeStruct((Q, NPAD), jnp.float32),
            jax.ShapeDtypeStruct((Q, 16), jnp.int32),
            jax.ShapeDtypeStruct((Q, 16), jnp.int32),
        ],
        scratch_shapes=[pltpu.VMEM((Q, NCHUNK), jnp.float32)],
    )(queries, keys_t)


def _run_k345(sims, bq16, lq16, token_bank):
    mesh = plsc.VectorSubcoreMesh(core_axis_name="c", subcore_axis_name="s")

    @functools.partial(
        pl.kernel,
        out_type=[
            jax.ShapeDtypeStruct((Q, D), jnp.float32),
            jax.ShapeDtypeStruct((Q, 16), jnp.int32),
        ],
        mesh=mesh,
        scratch_types=[
            pltpu.VMEM((16,), jnp.int32),
            pltpu.VMEM((16,), jnp.int32),
            pltpu.VMEM((2 * BLK,), jnp.float32),
            pltpu.VMEM((TOP_K, D, 128), jnp.float32),
            pltpu.VMEM((16,), jnp.int32),
            pltpu.VMEM((D,), jnp.float32),
            pltpu.SemaphoreType.DMA,
        ],
        compiler_params=_SC_PARAMS,
    )
    def k345(sims_hbm, bq_hbm, lq_hbm, tokt_hbm, prompt_hbm, idx_hbm,
             b_v, l_v, rows_v, tokslab_v, iout_v, acc_v, sem):
        qid = lax.axis_index("s") * NUM_SC_CORES + lax.axis_index("c")
        pltpu.sync_copy(bq_hbm.at[qid], b_v)
        pltpu.sync_copy(lq_hbm.at[qid], l_v)
        bvec = b_v[...]
        lvec = l_v[...]
        lane = lax.iota(jnp.int32, 16)

        cand = []
        pos = []
        for wave in range(5):
            copies = []
            for s in range(2):
                r = wave * 2 + s
                br = _extract(bvec, lane, r, -1)
                copies.append(pltpu.async_copy(
                    sims_hbm.at[qid, pl.ds(br * BLK, BLK)],
                    rows_v.at[pl.ds(s * BLK, BLK)], sem))
            for cp in copies:
                cp.wait()
            for s in range(2):
                r = wave * 2 + s
                br = _extract(bvec, lane, r, -1)
                lr = _extract(lvec, lane, r, -1)
                for c4 in range(BLK // 2048):
                    off = lr + 2048 * c4 + 128 * lane
                    cand.append(plsc.load_gather(rows_v, [s * BLK + off]))
                    pos.append(br * BLK + off)

        vals16 = jnp.full((16,), NEG, jnp.float32)
        idxs16 = jnp.zeros((16,), jnp.int32)
        for r in range(TOP_K):
            mvec = cand[0]
            for g in range(1, len(cand)):
                mvec = jnp.maximum(mvec, cand[g])
            m = jnp.max(mvec)
            pvec = jnp.where(cand[0] == m, pos[0], BIG)
            for g in range(1, len(cand)):
                pvec = jnp.minimum(pvec, jnp.where(cand[g] == m, pos[g], BIG))
            pi = jnp.min(pvec)
            vals16 = jnp.where(lane == r, m, vals16)
            idxs16 = jnp.where(lane == r, pi, idxs16)
            cand = [jnp.where(pos[g] == pi, NEG, cand[g])
                    for g in range(len(cand))]

        t = vals16 / TEMPERATURE
        e = jnp.exp(t - jnp.max(t))
        w = e / jnp.sum(e)

        iout_v[...] = idxs16
        pltpu.sync_copy(iout_v, idx_hbm.at[qid])

        copies = []
        for j in range(TOP_K):
            rj = _extract(idxs16, lane, j, -1)
            off = pl.multiple_of((rj >> 7) * 128, 128)
            copies.append(pltpu.async_copy(
                tokt_hbm.at[:, pl.ds(off, 128)], tokslab_v.at[j], sem))
        for cp in copies:
            cp.wait()
        for c in range(D // 16):
            acc = jnp.zeros((16,), jnp.float32)
            for j in range(TOP_K):
                rj = _extract(idxs16, lane, j, -1)
                rm = jnp.full((16,), rj & 127, jnp.int32)
                wj = _extract(w, lane, j, NEG)
                v = plsc.load_gather(
                    tokslab_v,
                    [jnp.full((16,), j, jnp.int32), lane + 16 * c, rm])
                acc = acc + v * wj
            acc_v[pl.ds(c * 16, 16)] = acc
        pltpu.sync_copy(acc_v, prompt_hbm.at[qid])

    return k345(sims, bq16, lq16, token_bank)


def kernel(queries, keys, token_bank):
    sims, bq16, lq16 = _run_k1(queries, keys.T)
    prompt, idx16 = _run_k345(sims, bq16, lq16, token_bank.T)
    top_idx = idx16[:, :TOP_K]
    return prompt, top_idx


# speedup vs baseline: 5.9707x; 1.0141x over previous
import functools

import jax
import jax.numpy as jnp
from jax import lax
from jax.experimental import pallas as pl
from jax.experimental.pallas import tpu as pltpu
from jax.experimental.pallas import tpu_sc as plsc

Q = 32
D = 64
N = 1_000_000
BLK = 32768
NB = (N + BLK - 1) // BLK
NPAD = NB * BLK
NCHUNK = NB * 128
TOP_K = 10
CAND = TOP_K * (BLK // 128)
TEMPERATURE = 0.07
NEG = float("-inf")
BIG = 2 ** 30

NUM_SC_CORES = 2
NUM_SC_SUBCORES = 16

_SC_PARAMS = pltpu.CompilerParams(needs_layout_passes=False)


def _extract(vec, lane, j, fill):
    return jnp.max(jnp.where(lane == j, vec, fill))


def _k1_body(q_ref, kt_ref, sims_ref, bq_ref, lq_ref, cmax_ref):
    b = pl.program_id(0)
    q = q_ref[...]
    qn = q / (jnp.sqrt(jnp.sum(q * q, axis=1, keepdims=True)) + 1e-8)
    qh = qn.astype(jnp.bfloat16)

    kt = kt_ref[...]
    ksq = kt * kt
    s2 = jnp.sum(ksq, axis=0, keepdims=True)
    rinv = 1.0 / (jnp.sqrt(s2) + 1e-8)
    kh = (kt * rinv).astype(jnp.bfloat16)
    sim = lax.dot_general(qh, kh, (((1,), (0,)), ((), ())),
                          preferred_element_type=jnp.float32)

    col = b * BLK + lax.broadcasted_iota(jnp.int32, (1, BLK), 1)
    sim = jnp.where(col < N, sim, NEG)
    sims_ref[...] = sim

    m = sim[:, 0:128]
    for c in range(1, BLK // 128):
        m = jnp.maximum(m, sim[:, c * 128:(c + 1) * 128])
    cmax_ref[:, pl.ds(pl.multiple_of(b * 128, 128), 128)] = m

    @pl.when(b == NB - 1)
    def _select():
        cm = cmax_ref[...]
        ids = lax.broadcasted_iota(jnp.int32, (Q, NCHUNK), 1)
        bq_ref[...] = jnp.zeros((Q, 16), jnp.int32)
        lq_ref[...] = jnp.zeros((Q, 16), jnp.int32)
        for r in range(TOP_K):
            mval = jnp.max(cm, axis=1, keepdims=True)
            sel = jnp.min(jnp.where(cm == mval, ids, BIG),
                          axis=1, keepdims=True)
            cm = jnp.where(ids == sel, NEG, cm)
            bq_ref[:, r:r + 1] = sel >> 7
            lq_ref[:, r:r + 1] = sel & 127


def _run_k1(queries, keys_t):
    return pl.pallas_call(
        _k1_body,
        grid=(NB,),
        in_specs=[
            pl.BlockSpec((Q, D), lambda b: (0, 0)),
            pl.BlockSpec((D, BLK), lambda b: (0, b)),
        ],
        out_specs=[
            pl.BlockSpec((Q, BLK), lambda b: (0, b)),
            pl.BlockSpec((Q, 16), lambda b: (0, 0)),
            pl.BlockSpec((Q, 16), lambda b: (0, 0)),
        ],
        out_shape=[
            jax.ShapeDtypeStruct((Q, NPAD), jnp.float32),
            jax.ShapeDtypeStruct((Q, 16), jnp.int32),
            jax.ShapeDtypeStruct((Q, 16), jnp.int32),
        ],
        scratch_shapes=[pltpu.VMEM((Q, NCHUNK), jnp.float32)],
    )(queries, keys_t)


def _run_k345(sims, bq16, lq16, token_bank):
    mesh = plsc.VectorSubcoreMesh(core_axis_name="c", subcore_axis_name="s")

    @functools.partial(
        pl.kernel,
        out_type=[
            jax.ShapeDtypeStruct((Q, D), jnp.float32),
            jax.ShapeDtypeStruct((Q, 16), jnp.int32),
        ],
        mesh=mesh,
        scratch_types=[
            pltpu.VMEM((16,), jnp.int32),
            pltpu.VMEM((16,), jnp.int32),
            pltpu.VMEM((2 * BLK,), jnp.float32),
            pltpu.VMEM((CAND,), jnp.float32),
            pltpu.VMEM((CAND,), jnp.int32),
            pltpu.VMEM((5, D, 128), jnp.float32),
            pltpu.VMEM((16,), jnp.int32),
            pltpu.VMEM((D,), jnp.float32),
            pltpu.SemaphoreType.DMA,
        ],
        compiler_params=_SC_PARAMS,
    )
    def k345(sims_hbm, bq_hbm, lq_hbm, tokt_hbm, prompt_hbm, idx_hbm,
             b_v, l_v, rows_v, cand_v, pos_v, tokslab_v, iout_v, acc_v, sem):
        qid = lax.axis_index("s") * NUM_SC_CORES + lax.axis_index("c")
        pltpu.sync_copy(bq_hbm.at[qid], b_v)
        pltpu.sync_copy(lq_hbm.at[qid], l_v)
        bvec = b_v[...]
        lvec = l_v[...]
        lane = lax.iota(jnp.int32, 16)

        def seg_copy(r, slot):
            br = _extract(bvec, lane, r, -1)
            return pltpu.async_copy(sims_hbm.at[qid, pl.ds(br * BLK, BLK)],
                                    rows_v.at[pl.ds(slot * BLK, BLK)], sem)

        G = BLK // 2048
        cps = [None] * TOP_K
        cps[0] = seg_copy(0, 0)
        for r in range(TOP_K):
            if r + 1 < TOP_K:
                cps[r + 1] = seg_copy(r + 1, (r + 1) % 2)
            cps[r].wait()
            br = _extract(bvec, lane, r, -1)
            lr = _extract(lvec, lane, r, -1)
            for c4 in range(G):
                off = lr + 2048 * c4 + 128 * lane
                g = r * G + c4
                cand_v[pl.ds(g * 16, 16)] = plsc.load_gather(
                    rows_v, [(r % 2) * BLK + off])
                pos_v[pl.ds(g * 16, 16)] = br * BLK + off

        ngrp = TOP_K * G

        def round_body(r, carry):
            vals16, idxs16 = carry
            mvec = cand_v[pl.ds(0, 16)]
            for g in range(1, ngrp):
                mvec = jnp.maximum(mvec, cand_v[pl.ds(g * 16, 16)])
            m = jnp.max(mvec)
            pvec = jnp.full((16,), BIG, jnp.int32)
            for g in range(ngrp):
                pvec = jnp.minimum(
                    pvec, jnp.where(cand_v[pl.ds(g * 16, 16)] == m,
                                    pos_v[pl.ds(g * 16, 16)], BIG))
            pi = jnp.min(pvec)
            for g in range(ngrp):
                cg = cand_v[pl.ds(g * 16, 16)]
                cand_v[pl.ds(g * 16, 16)] = jnp.where(
                    pos_v[pl.ds(g * 16, 16)] == pi, NEG, cg)
            vals16 = jnp.where(lane == r, m, vals16)
            idxs16 = jnp.where(lane == r, pi, idxs16)
            return vals16, idxs16

        vals16, idxs16 = lax.fori_loop(
            0, TOP_K,
            round_body,
            (jnp.full((16,), NEG, jnp.float32), jnp.zeros((16,), jnp.int32)))

        t = vals16 / TEMPERATURE
        e = jnp.exp(t - jnp.max(t))
        w = e / jnp.sum(e)

        iout_v[...] = idxs16
        pltpu.sync_copy(iout_v, idx_hbm.at[qid])

        accs = [jnp.zeros((16,), jnp.float32) for _ in range(D // 16)]
        for wave in range(2):
            copies = []
            for s in range(5):
                j = wave * 5 + s
                rj = _extract(idxs16, lane, j, -1)
                off = pl.multiple_of((rj >> 7) * 128, 128)
                copies.append(pltpu.async_copy(
                    tokt_hbm.at[:, pl.ds(off, 128)], tokslab_v.at[s], sem))
            for cp in copies:
                cp.wait()
            for s in range(5):
                j = wave * 5 + s
                rj = _extract(idxs16, lane, j, -1)
                rm = jnp.full((16,), rj & 127, jnp.int32)
                wj = _extract(w, lane, j, NEG)
                for c in range(D // 16):
                    v = plsc.load_gather(
                        tokslab_v,
                        [jnp.full((16,), s, jnp.int32), lane + 16 * c, rm])
                    accs[c] = accs[c] + v * wj
        for c in range(D // 16):
            acc_v[pl.ds(c * 16, 16)] = accs[c]
        pltpu.sync_copy(acc_v, prompt_hbm.at[qid])

    return k345(sims, bq16, lq16, token_bank)


def kernel(queries, keys, token_bank):
    sims, bq16, lq16 = _run_k1(queries, keys.T)
    prompt, idx16 = _run_k345(sims, bq16, lq16, token_bank.T)
    top_idx = idx16[:, :TOP_K]
    return prompt, top_idx


# speedup vs baseline: 6.3142x; 1.0575x over previous
import functools

import jax
import jax.numpy as jnp
from jax import lax
from jax.experimental import pallas as pl
from jax.experimental.pallas import tpu as pltpu
from jax.experimental.pallas import tpu_sc as plsc

Q = 32
D = 64
N = 1_000_000
BLK = 32768
NB = (N + BLK - 1) // BLK
NPAD = NB * BLK
SEG = 16384
NSEG = NPAD // SEG
NCHUNK = NSEG * 128
TOP_K = 10
CAND = TOP_K * (SEG // 128)
TEMPERATURE = 0.07
NEG = float("-inf")
BIG = 2 ** 30

NUM_SC_CORES = 2
NUM_SC_SUBCORES = 16

_SC_PARAMS = pltpu.CompilerParams(needs_layout_passes=False)


def _extract(vec, lane, j, fill):
    return jnp.max(jnp.where(lane == j, vec, fill))


def _k1_body(q_ref, kt_ref, sims_ref, bq_ref, lq_ref, cmax_ref):
    b = pl.program_id(0)
    q = q_ref[...]
    qn = q / (jnp.sqrt(jnp.sum(q * q, axis=1, keepdims=True)) + 1e-8)
    qh = qn.astype(jnp.bfloat16)

    kt = kt_ref[...]
    ksq = kt * kt
    s2 = jnp.sum(ksq, axis=0, keepdims=True)
    rinv = 1.0 / (jnp.sqrt(s2) + 1e-8)
    kh = (kt * rinv).astype(jnp.bfloat16)
    sim = lax.dot_general(qh, kh, (((1,), (0,)), ((), ())),
                          preferred_element_type=jnp.float32)

    col = b * BLK + lax.broadcasted_iota(jnp.int32, (1, BLK), 1)
    sim = jnp.where(col < N, sim, NEG)
    sims_ref[...] = sim

    for half in range(BLK // SEG):
        m = sim[:, half * SEG:half * SEG + 128]
        for c in range(1, SEG // 128):
            base = half * SEG + c * 128
            m = jnp.maximum(m, sim[:, base:base + 128])
        cmax_ref[:, pl.ds(pl.multiple_of((2 * b + half) * 128, 128), 128)] = m

    @pl.when(b == NB - 1)
    def _select():
        cm = cmax_ref[...]
        ids = lax.broadcasted_iota(jnp.int32, (Q, NCHUNK), 1)
        bq_ref[...] = jnp.zeros((Q, 16), jnp.int32)
        lq_ref[...] = jnp.zeros((Q, 16), jnp.int32)
        for r in range(TOP_K):
            mval = jnp.max(cm, axis=1, keepdims=True)
            sel = jnp.min(jnp.where(cm == mval, ids, BIG),
                          axis=1, keepdims=True)
            cm = jnp.where(ids == sel, NEG, cm)
            bq_ref[:, r:r + 1] = sel >> 7
            lq_ref[:, r:r + 1] = sel & 127


def _run_k1(queries, keys_t):
    return pl.pallas_call(
        _k1_body,
        grid=(NB,),
        in_specs=[
            pl.BlockSpec((Q, D), lambda b: (0, 0)),
            pl.BlockSpec((D, BLK), lambda b: (0, b)),
        ],
        out_specs=[
            pl.BlockSpec((Q, BLK), lambda b: (0, b)),
            pl.BlockSpec((Q, 16), lambda b: (0, 0)),
            pl.BlockSpec((Q, 16), lambda b: (0, 0)),
        ],
        out_shape=[
            jax.ShapeDtypeStruct((Q, NPAD), jnp.float32),
            jax.ShapeDtypeStruct((Q, 16), jnp.int32),
            jax.ShapeDtypeStruct((Q, 16), jnp.int32),
        ],
        scratch_shapes=[pltpu.VMEM((Q, NCHUNK), jnp.float32)],
    )(queries, keys_t)


def _run_k345(sims, bq16, lq16, token_bank):
    mesh = plsc.VectorSubcoreMesh(core_axis_name="c", subcore_axis_name="s")

    @functools.partial(
        pl.kernel,
        out_type=[
            jax.ShapeDtypeStruct((Q, D), jnp.float32),
            jax.ShapeDtypeStruct((Q, 16), jnp.int32),
        ],
        mesh=mesh,
        scratch_types=[
            pltpu.VMEM((16,), jnp.int32),
            pltpu.VMEM((16,), jnp.int32),
            pltpu.VMEM((2 * SEG,), jnp.float32),
            pltpu.VMEM((CAND,), jnp.float32),
            pltpu.VMEM((CAND,), jnp.int32),
            pltpu.VMEM((5, D, 128), jnp.float32),
            pltpu.VMEM((16,), jnp.int32),
            pltpu.VMEM((D,), jnp.float32),
            pltpu.SemaphoreType.DMA,
        ],
        compiler_params=_SC_PARAMS,
    )
    def k345(sims_hbm, bq_hbm, lq_hbm, tokt_hbm, prompt_hbm, idx_hbm,
             b_v, l_v, rows_v, cand_v, pos_v, tokslab_v, iout_v, acc_v, sem):
        qid = lax.axis_index("s") * NUM_SC_CORES + lax.axis_index("c")
        pltpu.sync_copy(bq_hbm.at[qid], b_v)
        pltpu.sync_copy(lq_hbm.at[qid], l_v)
        bvec = b_v[...]
        lvec = l_v[...]
        lane = lax.iota(jnp.int32, 16)

        def seg_copy(r, slot):
            br = _extract(bvec, lane, r, -1)
            return pltpu.async_copy(sims_hbm.at[qid, pl.ds(br * SEG, SEG)],
                                    rows_v.at[pl.ds(slot * SEG, SEG)], sem)

        G = SEG // 2048
        cps = [None] * TOP_K
        cps[0] = seg_copy(0, 0)
        for r in range(TOP_K):
            if r + 1 < TOP_K:
                cps[r + 1] = seg_copy(r + 1, (r + 1) % 2)
            cps[r].wait()
            br = _extract(bvec, lane, r, -1)
            lr = _extract(lvec, lane, r, -1)
            for c4 in range(G):
                off = lr + 2048 * c4 + 128 * lane
                g = r * G + c4
                cand_v[pl.ds(g * 16, 16)] = plsc.load_gather(
                    rows_v, [(r % 2) * SEG + off])
                pos_v[pl.ds(g * 16, 16)] = br * SEG + off

        ngrp = TOP_K * G

        def round_body(r, carry):
            vals16, idxs16 = carry
            mvec = cand_v[pl.ds(0, 16)]
            for g in range(1, ngrp):
                mvec = jnp.maximum(mvec, cand_v[pl.ds(g * 16, 16)])
            m = jnp.max(mvec)
            pvec = jnp.full((16,), BIG, jnp.int32)
            for g in range(ngrp):
                pvec = jnp.minimum(
                    pvec, jnp.where(cand_v[pl.ds(g * 16, 16)] == m,
                                    pos_v[pl.ds(g * 16, 16)], BIG))
            pi = jnp.min(pvec)
            for g in range(ngrp):
                cg = cand_v[pl.ds(g * 16, 16)]
                cand_v[pl.ds(g * 16, 16)] = jnp.where(
                    pos_v[pl.ds(g * 16, 16)] == pi, NEG, cg)
            vals16 = jnp.where(lane == r, m, vals16)
            idxs16 = jnp.where(lane == r, pi, idxs16)
            return vals16, idxs16

        vals16, idxs16 = lax.fori_loop(
            0, TOP_K,
            round_body,
            (jnp.full((16,), NEG, jnp.float32), jnp.zeros((16,), jnp.int32)))

        t = vals16 / TEMPERATURE
        e = jnp.exp(t - jnp.max(t))
        w = e / jnp.sum(e)

        iout_v[...] = idxs16
        pltpu.sync_copy(iout_v, idx_hbm.at[qid])

        accs = [jnp.zeros((16,), jnp.float32) for _ in range(D // 16)]
        for wave in range(2):
            copies = []
            for s in range(5):
                j = wave * 5 + s
                rj = _extract(idxs16, lane, j, -1)
                off = pl.multiple_of((rj >> 7) * 128, 128)
                copies.append(pltpu.async_copy(
                    tokt_hbm.at[:, pl.ds(off, 128)], tokslab_v.at[s], sem))
            for cp in copies:
                cp.wait()
            for s in range(5):
                j = wave * 5 + s
                rj = _extract(idxs16, lane, j, -1)
                rm = jnp.full((16,), rj & 127, jnp.int32)
                wj = _extract(w, lane, j, NEG)
                for c in range(D // 16):
                    v = plsc.load_gather(
                        tokslab_v,
                        [jnp.full((16,), s, jnp.int32), lane + 16 * c, rm])
                    accs[c] = accs[c] + v * wj
        for c in range(D // 16):
            acc_v[pl.ds(c * 16, 16)] = accs[c]
        pltpu.sync_copy(acc_v, prompt_hbm.at[qid])

    return k345(sims, bq16, lq16, token_bank)


def kernel(queries, keys, token_bank):
    sims, bq16, lq16 = _run_k1(queries, keys.T)
    prompt, idx16 = _run_k345(sims, bq16, lq16, token_bank.T)
    top_idx = idx16[:, :TOP_K]
    return prompt, top_idx


# speedup vs baseline: 6.4457x; 1.0208x over previous
import functools

import jax
import jax.numpy as jnp
from jax import lax
from jax.experimental import pallas as pl
from jax.experimental.pallas import tpu as pltpu
from jax.experimental.pallas import tpu_sc as plsc

Q = 32
D = 64
N = 1_000_000
BLK = 32768
NB = (N + BLK - 1) // BLK
NPAD = NB * BLK
SEG = 8192
NSEG = NPAD // SEG
NCHUNK = NSEG * 128
TOP_K = 10
CAND = TOP_K * (SEG // 128)
TEMPERATURE = 0.07
NEG = float("-inf")
BIG = 2 ** 30

NUM_SC_CORES = 2
NUM_SC_SUBCORES = 16

_SC_PARAMS = pltpu.CompilerParams(needs_layout_passes=False)


def _extract(vec, lane, j, fill):
    return jnp.max(jnp.where(lane == j, vec, fill))


def _k1_body(q_ref, kt_ref, sims_ref, bq_ref, lq_ref, cmax_ref):
    b = pl.program_id(0)
    q = q_ref[...]
    qn = q / (jnp.sqrt(jnp.sum(q * q, axis=1, keepdims=True)) + 1e-8)
    qh = qn.astype(jnp.bfloat16)

    kt = kt_ref[...]
    ksq = kt * kt
    s2 = jnp.sum(ksq, axis=0, keepdims=True)
    rinv = 1.0 / (jnp.sqrt(s2) + 1e-8)
    kh = (kt * rinv).astype(jnp.bfloat16)
    sim = lax.dot_general(qh, kh, (((1,), (0,)), ((), ())),
                          preferred_element_type=jnp.float32)

    col = b * BLK + lax.broadcasted_iota(jnp.int32, (1, BLK), 1)
    sim = jnp.where(col < N, sim, NEG)
    sims_ref[...] = sim

    for half in range(BLK // SEG):
        m = sim[:, half * SEG:half * SEG + 128]
        for c in range(1, SEG // 128):
            base = half * SEG + c * 128
            m = jnp.maximum(m, sim[:, base:base + 128])
        cmax_ref[:, pl.ds(pl.multiple_of((2 * b + half) * 128, 128), 128)] = m

    @pl.when(b == NB - 1)
    def _select():
        cm = cmax_ref[...]
        ids = lax.broadcasted_iota(jnp.int32, (Q, NCHUNK), 1)
        bq_ref[...] = jnp.zeros((Q, 16), jnp.int32)
        lq_ref[...] = jnp.zeros((Q, 16), jnp.int32)
        for r in range(TOP_K):
            mval = jnp.max(cm, axis=1, keepdims=True)
            sel = jnp.min(jnp.where(cm == mval, ids, BIG),
                          axis=1, keepdims=True)
            cm = jnp.where(ids == sel, NEG, cm)
            bq_ref[:, r:r + 1] = sel >> 7
            lq_ref[:, r:r + 1] = sel & 127


def _run_k1(queries, keys_t):
    return pl.pallas_call(
        _k1_body,
        grid=(NB,),
        in_specs=[
            pl.BlockSpec((Q, D), lambda b: (0, 0)),
            pl.BlockSpec((D, BLK), lambda b: (0, b)),
        ],
        out_specs=[
            pl.BlockSpec((Q, BLK), lambda b: (0, b)),
            pl.BlockSpec((Q, 16), lambda b: (0, 0)),
            pl.BlockSpec((Q, 16), lambda b: (0, 0)),
        ],
        out_shape=[
            jax.ShapeDtypeStruct((Q, NPAD), jnp.float32),
            jax.ShapeDtypeStruct((Q, 16), jnp.int32),
            jax.ShapeDtypeStruct((Q, 16), jnp.int32),
        ],
        scratch_shapes=[pltpu.VMEM((Q, NCHUNK), jnp.float32)],
    )(queries, keys_t)


def _run_k345(sims, bq16, lq16, token_bank):
    mesh = plsc.VectorSubcoreMesh(core_axis_name="c", subcore_axis_name="s")

    @functools.partial(
        pl.kernel,
        out_type=[
            jax.ShapeDtypeStruct((Q, D), jnp.float32),
            jax.ShapeDtypeStruct((Q, 16), jnp.int32),
        ],
        mesh=mesh,
        scratch_types=[
            pltpu.VMEM((16,), jnp.int32),
            pltpu.VMEM((16,), jnp.int32),
            pltpu.VMEM((2 * SEG,), jnp.float32),
            pltpu.VMEM((CAND,), jnp.float32),
            pltpu.VMEM((CAND,), jnp.int32),
            pltpu.VMEM((5, D, 128), jnp.float32),
            pltpu.VMEM((16,), jnp.int32),
            pltpu.VMEM((D,), jnp.float32),
            pltpu.SemaphoreType.DMA,
        ],
        compiler_params=_SC_PARAMS,
    )
    def k345(sims_hbm, bq_hbm, lq_hbm, tokt_hbm, prompt_hbm, idx_hbm,
             b_v, l_v, rows_v, cand_v, pos_v, tokslab_v, iout_v, acc_v, sem):
        qid = lax.axis_index("s") * NUM_SC_CORES + lax.axis_index("c")
        pltpu.sync_copy(bq_hbm.at[qid], b_v)
        pltpu.sync_copy(lq_hbm.at[qid], l_v)
        bvec = b_v[...]
        lvec = l_v[...]
        lane = lax.iota(jnp.int32, 16)

        def seg_copy(r, slot):
            br = _extract(bvec, lane, r, -1)
            return pltpu.async_copy(sims_hbm.at[qid, pl.ds(br * SEG, SEG)],
                                    rows_v.at[pl.ds(slot * SEG, SEG)], sem)

        G = SEG // 2048
        cps = [None] * TOP_K
        cps[0] = seg_copy(0, 0)
        for r in range(TOP_K):
            if r + 1 < TOP_K:
                cps[r + 1] = seg_copy(r + 1, (r + 1) % 2)
            cps[r].wait()
            br = _extract(bvec, lane, r, -1)
            lr = _extract(lvec, lane, r, -1)
            for c4 in range(G):
                off = lr + 2048 * c4 + 128 * lane
                g = r * G + c4
                cand_v[pl.ds(g * 16, 16)] = plsc.load_gather(
                    rows_v, [(r % 2) * SEG + off])
                pos_v[pl.ds(g * 16, 16)] = br * SEG + off

        ngrp = TOP_K * G

        def round_body(r, carry):
            vals16, idxs16 = carry
            mvec = cand_v[pl.ds(0, 16)]
            for g in range(1, ngrp):
                mvec = jnp.maximum(mvec, cand_v[pl.ds(g * 16, 16)])
            m = jnp.max(mvec)
            pvec = jnp.full((16,), BIG, jnp.int32)
            for g in range(ngrp):
                pvec = jnp.minimum(
                    pvec, jnp.where(cand_v[pl.ds(g * 16, 16)] == m,
                                    pos_v[pl.ds(g * 16, 16)], BIG))
            pi = jnp.min(pvec)
            for g in range(ngrp):
                cg = cand_v[pl.ds(g * 16, 16)]
                cand_v[pl.ds(g * 16, 16)] = jnp.where(
                    pos_v[pl.ds(g * 16, 16)] == pi, NEG, cg)
            vals16 = jnp.where(lane == r, m, vals16)
            idxs16 = jnp.where(lane == r, pi, idxs16)
            return vals16, idxs16

        vals16, idxs16 = lax.fori_loop(
            0, TOP_K,
            round_body,
            (jnp.full((16,), NEG, jnp.float32), jnp.zeros((16,), jnp.int32)))

        t = vals16 / TEMPERATURE
        e = jnp.exp(t - jnp.max(t))
        w = e / jnp.sum(e)

        iout_v[...] = idxs16
        pltpu.sync_copy(iout_v, idx_hbm.at[qid])

        accs = [jnp.zeros((16,), jnp.float32) for _ in range(D // 16)]
        for wave in range(2):
            copies = []
            for s in range(5):
                j = wave * 5 + s
                rj = _extract(idxs16, lane, j, -1)
                off = pl.multiple_of((rj >> 7) * 128, 128)
                copies.append(pltpu.async_copy(
                    tokt_hbm.at[:, pl.ds(off, 128)], tokslab_v.at[s], sem))
            for cp in copies:
                cp.wait()
            for s in range(5):
                j = wave * 5 + s
                rj = _extract(idxs16, lane, j, -1)
                rm = jnp.full((16,), rj & 127, jnp.int32)
                wj = _extract(w, lane, j, NEG)
                for c in range(D // 16):
                    v = plsc.load_gather(
                        tokslab_v,
                        [jnp.full((16,), s, jnp.int32), lane + 16 * c, rm])
                    accs[c] = accs[c] + v * wj
        for c in range(D // 16):
            acc_v[pl.ds(c * 16, 16)] = accs[c]
        pltpu.sync_copy(acc_v, prompt_hbm.at[qid])

    return k345(sims, bq16, lq16, token_bank)


def kernel(queries, keys, token_bank):
    sims, bq16, lq16 = _run_k1(queries, keys.T)
    prompt, idx16 = _run_k345(sims, bq16, lq16, token_bank.T)
    top_idx = idx16[:, :TOP_K]
    return prompt, top_idx
